# baseline (speedup 1.0000x reference)
import functools

import jax
import jax.numpy as jnp
import numpy as np
from jax.experimental import pallas as pl
from jax.experimental.pallas import tpu as pltpu
from jax.experimental.pallas import tpu_sc as plsc

D_MODEL = 512
KNN = 16
_RSQRT_D = 1.0 / np.sqrt(D_MODEL).astype(np.float32)


def _index_points(points, idx):
    b = points.shape[0]
    batch = jnp.arange(b).reshape((b,) + (1,) * (idx.ndim - 1))
    return points[batch, idx]


def _square_distance(src, dst):
    d = -2.0 * jnp.einsum('bnc,bmc->bnm', src, dst)
    d = d + jnp.sum(src ** 2, -1)[:, :, None]
    d = d + jnp.sum(dst ** 2, -1)[:, None, :]
    return d


def _bn_train(x, p, axes):
    m = jnp.mean(x, axis=axes, keepdims=True)
    v = jnp.var(x, axis=axes, keepdims=True)
    return (x - m) / jnp.sqrt(v + 1e-5) * p['g'] + p['b']


def _linear(p, x):
    y = x @ p['w']
    if 'b' in p:
        y = y + p['b']
    return y


def _topk_body(kk, d_ref, o_ref):
    d = d_ref[...]
    p, m = d.shape
    iota = jax.lax.broadcasted_iota(jnp.int32, (p, m), 1)
    col = jax.lax.broadcasted_iota(jnp.int32, (p, kk), 1)
    idxacc = jnp.zeros((p, kk), jnp.int32)
    for k in range(kk):
        mn = jnp.min(d, axis=1, keepdims=True)
        am = jnp.min(jnp.where(d == mn, iota, m), axis=1, keepdims=True)
        idxacc = jnp.where(col == k, am.astype(jnp.int32), idxacc)
        d = jnp.where(iota == am, jnp.inf, d)
    o_ref[...] = idxacc


def _topk_one(dists, kk):
    n, m = dists.shape
    p = min(n, 256)
    return pl.pallas_call(
        functools.partial(_topk_body, kk),
        grid=(n // p,),
        in_specs=[pl.BlockSpec((p, m), lambda i: (i, 0))],
        out_specs=pl.BlockSpec((p, kk), lambda i: (i, 0)),
        out_shape=jax.ShapeDtypeStruct((n, kk), jnp.int32),
    )(dists)


def _ksmallest(dists, kk):
    return jax.vmap(lambda d: _topk_one(d, kk))(dists)


def _fps_body(npoint, xt_ref, xs_ref, o_ref):
    n = xt_ref.shape[1]
    iota = jax.lax.broadcasted_iota(jnp.int32, (1, n), 1)
    xr = xt_ref[0:1, :]
    yr = xt_ref[1:2, :]
    zr = xt_ref[2:3, :]

    def body(i, carry):
        distance, far = carry
        o_ref[0, i] = far
        cx = xs_ref[0, far]
        cy = xs_ref[0, n + far]
        cz = xs_ref[0, 2 * n + far]
        d = (xr - cx) ** 2 + (yr - cy) ** 2 + (zr - cz) ** 2
        distance = jnp.minimum(distance, d)
        mx = jnp.max(distance)
        far = jnp.min(jnp.where(distance == mx, iota, n)).astype(jnp.int32)
        return distance, far

    jax.lax.fori_loop(0, npoint, body,
                      (jnp.full((1, n), 1e10, jnp.float32), jnp.int32(0)))


def _fps_one(xyz, npoint):
    xt = xyz.T
    return pl.pallas_call(
        functools.partial(_fps_body, npoint),
        in_specs=[
            pl.BlockSpec(memory_space=pltpu.VMEM),
            pl.BlockSpec(memory_space=pltpu.SMEM),
        ],
        out_specs=pl.BlockSpec(memory_space=pltpu.SMEM),
        out_shape=jax.ShapeDtypeStruct((1, npoint), jnp.int32),
    )(xt, xt.reshape(1, -1))


def _farthest_point_sample(xyz, npoint):
    return jax.vmap(lambda x: _fps_one(x, npoint))(xyz)[:, 0, :]


def _mm(a, b):
    return jnp.dot(a.astype(jnp.bfloat16), b.astype(jnp.bfloat16),
                   preferred_element_type=jnp.float32)


_KV_D = 2 * D_MODEL + 128


def _proj_body(f_ref, xyz_ref, w1_ref, b1_ref, wq_ref, wkv_ref,
               q_ref, kv_ref):
    x = _mm(f_ref[...], w1_ref[...]) + b1_ref[...]
    q_ref[...] = _mm(x, wq_ref[...])
    kv_ref[:, :2 * D_MODEL] = _mm(x, wkv_ref[...])
    n = xyz_ref.shape[0]
    xpad = jnp.concatenate(
        [xyz_ref[...], jnp.zeros((n, 125), jnp.float32)], axis=1)
    kv_ref[:, 2 * D_MODEL:] = xpad


def _proj_one(feats, xyz, w1, b1, wq, wkv):
    n = feats.shape[0]
    return pl.pallas_call(
        _proj_body,
        out_shape=(jax.ShapeDtypeStruct((n, D_MODEL), jnp.float32),
                   jax.ShapeDtypeStruct((n, _KV_D), jnp.float32)),
    )(feats, xyz, w1, b1, wq, wkv)


_SC_NW = 32


def _sc_gather_rows(table, indices):
    m = indices.shape[0]
    d = table.shape[1]
    if m % (8 * _SC_NW) != 0:
        return jnp.take(table, indices, axis=0)
    b_per_w = m // _SC_NW
    ch = min(64, b_per_w)
    nchunk = b_per_w // ch
    mesh = plsc.VectorSubcoreMesh(core_axis_name="c", subcore_axis_name="s")

    @functools.partial(
        pl.kernel, mesh=mesh,
        out_type=jax.ShapeDtypeStruct((m, d), table.dtype),
        scratch_types=[
            pltpu.VMEM((ch,), jnp.int32),
            pltpu.VMEM((ch, d), table.dtype),
            pltpu.SemaphoreType.DMA,
        ])
    def kern(table_hbm, idx_hbm, out_hbm, idx_v, rows_v, sem):
        wid = jax.lax.axis_index("s") * 2 + jax.lax.axis_index("c")
        base = wid * b_per_w

        @pl.loop(0, nchunk)
        def _(c):
            off = base + c * ch
            pltpu.sync_copy(idx_hbm.at[pl.ds(off, ch)], idx_v)
            pltpu.async_copy(table_hbm.at[idx_v], rows_v, sem).wait()
            pltpu.sync_copy(rows_v, out_hbm.at[pl.ds(off, ch)])

    return kern(table, indices)


def _attn_body(q_ref, kvg_ref, xyz_ref, pre_ref,
               wd1_ref, bd1_ref, wd2_ref, bd2_ref,
               wg1_ref, bg1_ref, wg2_ref, bg2_ref,
               wf_ref, bf_ref, o_ref):
    kk, p, _ = kvg_ref.shape
    dm = D_MODEL
    qb = q_ref[...]
    xb = xyz_ref[...]
    kvtab = kvg_ref[...].reshape(kk * p, _KV_D)
    ktab = kvtab[:, :dm]
    vtab = kvtab[:, dm:2 * dm]
    xgv = kvtab[:, 2 * dm:2 * dm + 3]
    dx = jnp.broadcast_to(xb[None], (kk, p, 3)).reshape(kk * p, 3) - xgv
    r1 = jnp.maximum(_mm(dx, wd1_ref[...]) + bd1_ref[...], 0.0)
    pos = _mm(r1, wd2_ref[...]) + bd2_ref[...]
    qrep = jnp.broadcast_to(qb[None], (kk, p, dm)).reshape(kk * p, dm)
    h = qrep - ktab + pos
    sl = (_mm(jnp.maximum(_mm(h, wg1_ref[...]) + bg1_ref[...], 0.0),
              wg2_ref[...]) + bg2_ref[...]) * _RSQRT_D
    pv = vtab + pos
    m = sl[0:p]
    for k in range(1, kk):
        m = jnp.maximum(m, sl[k * p:(k + 1) * p])
    den = jnp.zeros_like(m)
    acc = jnp.zeros_like(m)
    for k in range(kk):
        e = jnp.exp(sl[k * p:(k + 1) * p] - m)
        den = den + e
        acc = acc + e * pv[k * p:(k + 1) * p]
    res = acc / den
    o_ref[...] = _mm(res, wf_ref[...]) + bf_ref[...] + pre_ref[...]


def _attn_one(q, kvg, xyz, pre,
              wd1, bd1, wd2, bd2, wg1, bg1, wg2, bg2, wf, bf):
    n = q.shape[0]
    kk = kvg.shape[0]
    dout = pre.shape[-1]
    p = min(n, 128)
    grid = (n // p,)
    full = lambda i: (0, 0)
    return pl.pallas_call(
        _attn_body,
        grid=grid,
        in_specs=[
            pl.BlockSpec((p, D_MODEL), lambda i: (i, 0)),
            pl.BlockSpec((kk, p, _KV_D), lambda i: (0, i, 0)),
            pl.BlockSpec((p, 3), lambda i: (i, 0)),
            pl.BlockSpec((p, dout), lambda i: (i, 0)),
            pl.BlockSpec((3, D_MODEL), full),
            pl.BlockSpec((1, D_MODEL), full),
            pl.BlockSpec((D_MODEL, D_MODEL), full),
            pl.BlockSpec((1, D_MODEL), full),
            pl.BlockSpec((D_MODEL, D_MODEL), full),
            pl.BlockSpec((1, D_MODEL), full),
            pl.BlockSpec((D_MODEL, D_MODEL), full),
            pl.BlockSpec((1, D_MODEL), full),
            pl.BlockSpec((D_MODEL, dout), full),
            pl.BlockSpec((1, dout), full),
        ],
        out_specs=pl.BlockSpec((p, dout), lambda i: (i, 0)),
        out_shape=jax.ShapeDtypeStruct((n, dout), jnp.float32),
    )(q, kvg, xyz, pre,
      wd1, bd1, wd2, bd2, wg1, bg1, wg2, bg2, wf, bf)


def _row(v):
    return v.reshape(1, -1)


def _transformer_block(p, xyz, feats):
    b, n, _ = xyz.shape
    kk = min(KNN, n)
    dists = _square_distance(xyz, xyz)
    idx = _ksmallest(dists, kk)
    idx_t = jnp.swapaxes(idx, 1, 2)

    proj = jax.vmap(_proj_one, in_axes=(0, 0, None, None, None, None))
    wkv = jnp.concatenate([p['wk']['w'], p['wv']['w']], axis=1)
    q, kvt = proj(feats, xyz, p['fc1']['w'], _row(p['fc1']['b']),
                  p['wq']['w'], wkv)

    flat_idx = (idx_t + (jnp.arange(b) * n)[:, None, None]).reshape(-1)
    kvg = _sc_gather_rows(kvt.reshape(b * n, _KV_D), flat_idx)
    kvg = kvg.reshape(b, kk, n, _KV_D)

    attn = jax.vmap(
        _attn_one,
        in_axes=(0, 0, 0, 0) + (None,) * 10)
    out = attn(q, kvg, xyz, feats,
               p['d1']['w'], _row(p['d1']['b']),
               p['d2']['w'], _row(p['d2']['b']),
               p['g1']['w'], _row(p['g1']['b']),
               p['g2']['w'], _row(p['g2']['b']),
               p['fc2']['w'], _row(p['fc2']['b']))
    return out


def _transition_down(p, xyz, points, npoint, nsample):
    fps_idx = _farthest_point_sample(xyz, npoint)
    new_xyz = _index_points(xyz, fps_idx)
    dists = _square_distance(new_xyz, xyz)
    idx = _ksmallest(dists, nsample)
    grouped_xyz = _index_points(xyz, idx)
    grouped_norm = grouped_xyz - new_xyz[:, :, None, :]
    grouped_pts = _index_points(points, idx)
    h = jnp.concatenate([grouped_norm, grouped_pts], axis=-1)
    h = jax.nn.relu(_bn_train(_linear(p['c1'], h), p['bn1'], (0, 1, 2)))
    h = jax.nn.relu(_bn_train(_linear(p['c2'], h), p['bn2'], (0, 1, 2)))
    return new_xyz, jnp.max(h, axis=2)


def _transition_up(p, xyz1, points1, xyz2, points2):
    feats1 = jax.nn.relu(_bn_train(_linear(p['fc1'], points1), p['bn1'], (0, 1)))
    feats2 = jax.nn.relu(_bn_train(_linear(p['fc2'], points2), p['bn2'], (0, 1)))
    dists = _square_distance(xyz2, xyz1)
    idx = _ksmallest(dists, 3)
    d3 = jnp.take_along_axis(dists, idx, axis=-1)
    recip = 1.0 / (d3 + 1e-8)
    w = recip / jnp.sum(recip, -1, keepdims=True)
    interp = jnp.sum(_index_points(feats1, idx) * w[..., None], axis=2)
    return interp + feats2


def _forward(params, x):
    nblocks = 4
    npts = x.shape[1]
    xyz = x[..., :3]
    h = _linear(params['bb_fc1b'], jax.nn.relu(_linear(params['bb_fc1a'], x)))
    points = _transformer_block(params['tf1'], xyz, h)
    xyz_and_feats = [(xyz, points)]
    for i in range(nblocks):
        xyz, points = _transition_down(params['td%d' % i], xyz, points,
                                       npts // 4 ** (i + 1), KNN)
        points = _transformer_block(params['bbtf%d' % i], xyz, points)
        xyz_and_feats.append((xyz, points))
    xyz = xyz_and_feats[-1][0]
    h = jax.nn.relu(_linear(params['f2a'], points))
    h = jax.nn.relu(_linear(params['f2b'], h))
    h = _linear(params['f2c'], h)
    points = _transformer_block(params['tf2'], xyz, h)
    for i in range(nblocks):
        points = _transition_up(params['tu%d' % i], xyz, points,
                                xyz_and_feats[-i - 2][0],
                                xyz_and_feats[-i - 2][1])
        xyz = xyz_and_feats[-i - 2][0]
        points = _transformer_block(params['uptf%d' % i], xyz, points)
    h = jax.nn.relu(_linear(params['f3a'], points))
    h = jax.nn.relu(_linear(params['f3b'], h))
    return _linear(params['f3c'], h)


def kernel(x, params):
    return _forward(params, x)


# speedup vs baseline: 1.3770x; 1.3770x over previous
import functools

import jax
import jax.numpy as jnp
import numpy as np
from jax.experimental import pallas as pl
from jax.experimental.pallas import tpu as pltpu
from jax.experimental.pallas import tpu_sc as plsc

D_MODEL = 512
KNN = 16
_RSQRT_D = 1.0 / np.sqrt(D_MODEL).astype(np.float32)


def _index_points(points, idx):
    b = points.shape[0]
    batch = jnp.arange(b).reshape((b,) + (1,) * (idx.ndim - 1))
    return points[batch, idx]


def _square_distance(src, dst):
    d = -2.0 * jnp.einsum('bnc,bmc->bnm', src, dst)
    d = d + jnp.sum(src ** 2, -1)[:, :, None]
    d = d + jnp.sum(dst ** 2, -1)[:, None, :]
    return d


def _bn_train(x, p, axes):
    m = jnp.mean(x, axis=axes, keepdims=True)
    v = jnp.var(x, axis=axes, keepdims=True)
    return (x - m) / jnp.sqrt(v + 1e-5) * p['g'] + p['b']


def _linear(p, x):
    y = x @ p['w']
    if 'b' in p:
        y = y + p['b']
    return y


def _topk_body(kk, d_ref, o_ref):
    d = d_ref[...]
    p, m = d.shape
    iota = jax.lax.broadcasted_iota(jnp.int32, (p, m), 1)
    col = jax.lax.broadcasted_iota(jnp.int32, (p, kk), 1)
    idxacc = jnp.zeros((p, kk), jnp.int32)
    for k in range(kk):
        mn = jnp.min(d, axis=1, keepdims=True)
        am = jnp.min(jnp.where(d == mn, iota, m), axis=1, keepdims=True)
        idxacc = jnp.where(col == k, am.astype(jnp.int32), idxacc)
        d = jnp.where(iota == am, jnp.inf, d)
    o_ref[...] = idxacc


def _topk_one(dists, kk):
    n, m = dists.shape
    p = min(n, 256)
    return pl.pallas_call(
        functools.partial(_topk_body, kk),
        grid=(n // p,),
        in_specs=[pl.BlockSpec((p, m), lambda i: (i, 0))],
        out_specs=pl.BlockSpec((p, kk), lambda i: (i, 0)),
        out_shape=jax.ShapeDtypeStruct((n, kk), jnp.int32),
    )(dists)


def _ksmallest(dists, kk):
    return jax.vmap(lambda d: _topk_one(d, kk))(dists)


def _fps_body(npoint, xt_ref, xs_ref, o_ref):
    n = xt_ref.shape[1]
    iota = jax.lax.broadcasted_iota(jnp.int32, (1, n), 1)
    xr = xt_ref[0:1, :]
    yr = xt_ref[1:2, :]
    zr = xt_ref[2:3, :]

    def body(i, carry):
        distance, far = carry
        o_ref[0, i] = far
        cx = xs_ref[0, far]
        cy = xs_ref[0, n + far]
        cz = xs_ref[0, 2 * n + far]
        d = (xr - cx) ** 2 + (yr - cy) ** 2 + (zr - cz) ** 2
        distance = jnp.minimum(distance, d)
        mx = jnp.max(distance)
        far = jnp.min(jnp.where(distance == mx, iota, n)).astype(jnp.int32)
        return distance, far

    jax.lax.fori_loop(0, npoint, body,
                      (jnp.full((1, n), 1e10, jnp.float32), jnp.int32(0)))


def _fps_one(xyz, npoint):
    xt = xyz.T
    return pl.pallas_call(
        functools.partial(_fps_body, npoint),
        in_specs=[
            pl.BlockSpec(memory_space=pltpu.VMEM),
            pl.BlockSpec(memory_space=pltpu.SMEM),
        ],
        out_specs=pl.BlockSpec(memory_space=pltpu.SMEM),
        out_shape=jax.ShapeDtypeStruct((1, npoint), jnp.int32),
    )(xt, xt.reshape(1, -1))


def _farthest_point_sample(xyz, npoint):
    return jax.vmap(lambda x: _fps_one(x, npoint))(xyz)[:, 0, :]


def _mm(a, b):
    return jnp.dot(a.astype(jnp.bfloat16), b.astype(jnp.bfloat16),
                   preferred_element_type=jnp.float32)


_KV_D = 2 * D_MODEL + 128


def _proj_body(f_ref, xyz_ref, w1_ref, b1_ref, wq_ref, wkv_ref,
               q_ref, kv_ref):
    x = _mm(f_ref[...], w1_ref[...]) + b1_ref[...]
    q_ref[...] = _mm(x, wq_ref[...])
    kv_ref[:, :2 * D_MODEL] = _mm(x, wkv_ref[...])
    n = xyz_ref.shape[0]
    xpad = jnp.concatenate(
        [xyz_ref[...], jnp.zeros((n, 125), jnp.float32)], axis=1)
    kv_ref[:, 2 * D_MODEL:] = xpad


def _proj_one(feats, xyz, w1, b1, wq, wkv):
    n = feats.shape[0]
    return pl.pallas_call(
        _proj_body,
        out_shape=(jax.ShapeDtypeStruct((n, D_MODEL), jnp.float32),
                   jax.ShapeDtypeStruct((n, _KV_D), jnp.float32)),
    )(feats, xyz, w1, b1, wq, wkv)


_SC_NW = 32


def _sc_gather_rows(table, indices):
    m = indices.shape[0]
    d = table.shape[1]
    if m % (8 * _SC_NW) != 0:
        return jnp.take(table, indices, axis=0)
    b_per_w = m // _SC_NW
    ch = min(64, b_per_w)
    nchunk = b_per_w // ch
    mesh = plsc.VectorSubcoreMesh(core_axis_name="c", subcore_axis_name="s")

    @functools.partial(
        pl.kernel, mesh=mesh,
        out_type=jax.ShapeDtypeStruct((m, d), table.dtype),
        scratch_types=[
            pltpu.VMEM((ch,), jnp.int32),
            pltpu.VMEM((ch, d), table.dtype),
            pltpu.SemaphoreType.DMA,
        ])
    def kern(table_hbm, idx_hbm, out_hbm, idx_v, rows_v, sem):
        wid = jax.lax.axis_index("s") * 2 + jax.lax.axis_index("c")
        base = wid * b_per_w

        @pl.loop(0, nchunk)
        def _(c):
            off = base + c * ch
            pltpu.sync_copy(idx_hbm.at[pl.ds(off, ch)], idx_v)
            pltpu.async_copy(table_hbm.at[idx_v], rows_v, sem).wait()
            pltpu.sync_copy(rows_v, out_hbm.at[pl.ds(off, ch)])

    return kern(table, indices)


def _pad128(x):
    c = x.shape[-1]
    pad = (-c) % 128
    if pad:
        x = jnp.concatenate([x, jnp.zeros(x.shape[:-1] + (pad,), x.dtype)], -1)
    return x


def _sc_gather_batched(table, idx):
    b, n, d = table.shape
    off = (jnp.arange(b, dtype=idx.dtype) * n).reshape((b,) + (1,) * (idx.ndim - 1))
    flat = (idx + off).reshape(-1)
    out = _sc_gather_rows(table.reshape(b * n, d), flat)
    return out.reshape(idx.shape + (d,))


def _attn_body(q_ref, kvg_ref, xyz_ref, pre_ref,
               wd1_ref, bd1_ref, wd2_ref, bd2_ref,
               wg1_ref, bg1_ref, wg2_ref, bg2_ref,
               wf_ref, bf_ref, o_ref):
    kk, p, _ = kvg_ref.shape
    dm = D_MODEL
    qb = q_ref[...]
    xb = xyz_ref[...]
    kvtab = kvg_ref[...].reshape(kk * p, _KV_D)
    ktab = kvtab[:, :dm]
    vtab = kvtab[:, dm:2 * dm]
    xgv = kvtab[:, 2 * dm:2 * dm + 3]
    dx = jnp.broadcast_to(xb[None], (kk, p, 3)).reshape(kk * p, 3) - xgv
    r1 = jnp.maximum(_mm(dx, wd1_ref[...]) + bd1_ref[...], 0.0)
    pos = _mm(r1, wd2_ref[...]) + bd2_ref[...]
    qrep = jnp.broadcast_to(qb[None], (kk, p, dm)).reshape(kk * p, dm)
    h = qrep - ktab + pos
    sl = (_mm(jnp.maximum(_mm(h, wg1_ref[...]) + bg1_ref[...], 0.0),
              wg2_ref[...]) + bg2_ref[...]) * _RSQRT_D
    pv = vtab + pos
    m = sl[0:p]
    for k in range(1, kk):
        m = jnp.maximum(m, sl[k * p:(k + 1) * p])
    den = jnp.zeros_like(m)
    acc = jnp.zeros_like(m)
    for k in range(kk):
        e = jnp.exp(sl[k * p:(k + 1) * p] - m)
        den = den + e
        acc = acc + e * pv[k * p:(k + 1) * p]
    res = acc / den
    o_ref[...] = _mm(res, wf_ref[...]) + bf_ref[...] + pre_ref[...]


def _attn_one(q, kvg, xyz, pre,
              wd1, bd1, wd2, bd2, wg1, bg1, wg2, bg2, wf, bf):
    n = q.shape[0]
    kk = kvg.shape[0]
    dout = pre.shape[-1]
    p = min(n, 128)
    grid = (n // p,)
    full = lambda i: (0, 0)
    return pl.pallas_call(
        _attn_body,
        grid=grid,
        in_specs=[
            pl.BlockSpec((p, D_MODEL), lambda i: (i, 0)),
            pl.BlockSpec((kk, p, _KV_D), lambda i: (0, i, 0)),
            pl.BlockSpec((p, 3), lambda i: (i, 0)),
            pl.BlockSpec((p, dout), lambda i: (i, 0)),
            pl.BlockSpec((3, D_MODEL), full),
            pl.BlockSpec((1, D_MODEL), full),
            pl.BlockSpec((D_MODEL, D_MODEL), full),
            pl.BlockSpec((1, D_MODEL), full),
            pl.BlockSpec((D_MODEL, D_MODEL), full),
            pl.BlockSpec((1, D_MODEL), full),
            pl.BlockSpec((D_MODEL, D_MODEL), full),
            pl.BlockSpec((1, D_MODEL), full),
            pl.BlockSpec((D_MODEL, dout), full),
            pl.BlockSpec((1, dout), full),
        ],
        out_specs=pl.BlockSpec((p, dout), lambda i: (i, 0)),
        out_shape=jax.ShapeDtypeStruct((n, dout), jnp.float32),
    )(q, kvg, xyz, pre,
      wd1, bd1, wd2, bd2, wg1, bg1, wg2, bg2, wf, bf)


def _row(v):
    return v.reshape(1, -1)


def _transformer_block(p, xyz, feats):
    b, n, _ = xyz.shape
    kk = min(KNN, n)
    dists = _square_distance(xyz, xyz)
    idx = _ksmallest(dists, kk)
    idx_t = jnp.swapaxes(idx, 1, 2)

    proj = jax.vmap(_proj_one, in_axes=(0, 0, None, None, None, None))
    wkv = jnp.concatenate([p['wk']['w'], p['wv']['w']], axis=1)
    q, kvt = proj(feats, xyz, p['fc1']['w'], _row(p['fc1']['b']),
                  p['wq']['w'], wkv)

    kvg = _sc_gather_batched(kvt, idx_t)

    attn = jax.vmap(
        _attn_one,
        in_axes=(0, 0, 0, 0) + (None,) * 10)
    out = attn(q, kvg, xyz, feats,
               p['d1']['w'], _row(p['d1']['b']),
               p['d2']['w'], _row(p['d2']['b']),
               p['g1']['w'], _row(p['g1']['b']),
               p['g2']['w'], _row(p['g2']['b']),
               p['fc2']['w'], _row(p['fc2']['b']))
    return out


def _transition_down(p, xyz, points, npoint, nsample):
    c = points.shape[-1]
    fps_idx = _farthest_point_sample(xyz, npoint)
    tbl = _pad128(jnp.concatenate([xyz, points], axis=-1))
    new_xyz = _sc_gather_batched(tbl, fps_idx)[..., :3]
    dists = _square_distance(new_xyz, xyz)
    idx = _ksmallest(dists, nsample)
    g = _sc_gather_batched(tbl, idx)
    grouped_norm = g[..., :3] - new_xyz[:, :, None, :]
    grouped_pts = g[..., 3:3 + c]
    h = jnp.concatenate([grouped_norm, grouped_pts], axis=-1)
    h = jax.nn.relu(_bn_train(_linear(p['c1'], h), p['bn1'], (0, 1, 2)))
    h = jax.nn.relu(_bn_train(_linear(p['c2'], h), p['bn2'], (0, 1, 2)))
    return new_xyz, jnp.max(h, axis=2)


def _transition_up(p, xyz1, points1, xyz2, points2):
    feats1 = jax.nn.relu(_bn_train(_linear(p['fc1'], points1), p['bn1'], (0, 1)))
    feats2 = jax.nn.relu(_bn_train(_linear(p['fc2'], points2), p['bn2'], (0, 1)))
    dists = _square_distance(xyz2, xyz1)
    idx = _ksmallest(dists, 3)
    d3 = jnp.take_along_axis(dists, idx, axis=-1)
    recip = 1.0 / (d3 + 1e-8)
    w = recip / jnp.sum(recip, -1, keepdims=True)
    c = feats1.shape[-1]
    rows = _sc_gather_batched(_pad128(feats1), idx)[..., :c]
    interp = jnp.sum(rows * w[..., None], axis=2)
    return interp + feats2


def _forward(params, x):
    nblocks = 4
    npts = x.shape[1]
    xyz = x[..., :3]
    h = _linear(params['bb_fc1b'], jax.nn.relu(_linear(params['bb_fc1a'], x)))
    points = _transformer_block(params['tf1'], xyz, h)
    xyz_and_feats = [(xyz, points)]
    for i in range(nblocks):
        xyz, points = _transition_down(params['td%d' % i], xyz, points,
                                       npts // 4 ** (i + 1), KNN)
        points = _transformer_block(params['bbtf%d' % i], xyz, points)
        xyz_and_feats.append((xyz, points))
    xyz = xyz_and_feats[-1][0]
    h = jax.nn.relu(_linear(params['f2a'], points))
    h = jax.nn.relu(_linear(params['f2b'], h))
    h = _linear(params['f2c'], h)
    points = _transformer_block(params['tf2'], xyz, h)
    for i in range(nblocks):
        points = _transition_up(params['tu%d' % i], xyz, points,
                                xyz_and_feats[-i - 2][0],
                                xyz_and_feats[-i - 2][1])
        xyz = xyz_and_feats[-i - 2][0]
        points = _transformer_block(params['uptf%d' % i], xyz, points)
    h = jax.nn.relu(_linear(params['f3a'], points))
    h = jax.nn.relu(_linear(params['f3b'], h))
    return _linear(params['f3c'], h)


def kernel(x, params):
    return _forward(params, x)


# speedup vs baseline: 1.3787x; 1.0013x over previous
import functools

import jax
import jax.numpy as jnp
import numpy as np
from jax.experimental import pallas as pl
from jax.experimental.pallas import tpu as pltpu
from jax.experimental.pallas import tpu_sc as plsc

D_MODEL = 512
KNN = 16
_RSQRT_D = 1.0 / np.sqrt(D_MODEL).astype(np.float32)


def _index_points(points, idx):
    b = points.shape[0]
    batch = jnp.arange(b).reshape((b,) + (1,) * (idx.ndim - 1))
    return points[batch, idx]


def _square_distance(src, dst):
    d = -2.0 * jnp.einsum('bnc,bmc->bnm', src, dst)
    d = d + jnp.sum(src ** 2, -1)[:, :, None]
    d = d + jnp.sum(dst ** 2, -1)[:, None, :]
    return d


def _bn_train(x, p, axes):
    m = jnp.mean(x, axis=axes, keepdims=True)
    v = jnp.var(x, axis=axes, keepdims=True)
    return (x - m) / jnp.sqrt(v + 1e-5) * p['g'] + p['b']


def _linear(p, x):
    y = x @ p['w']
    if 'b' in p:
        y = y + p['b']
    return y


def _topk_body(kk, d_ref, o_ref):
    d = d_ref[...]
    p, m = d.shape
    iota = jax.lax.broadcasted_iota(jnp.int32, (p, m), 1)
    col = jax.lax.broadcasted_iota(jnp.int32, (p, kk), 1)
    idxacc = jnp.zeros((p, kk), jnp.int32)
    for k in range(kk):
        mn = jnp.min(d, axis=1, keepdims=True)
        am = jnp.min(jnp.where(d == mn, iota, m), axis=1, keepdims=True)
        idxacc = jnp.where(col == k, am.astype(jnp.int32), idxacc)
        d = jnp.where(iota == am, jnp.inf, d)
    o_ref[...] = idxacc


def _topk_one(dists, kk):
    n, m = dists.shape
    p = min(n, 256)
    return pl.pallas_call(
        functools.partial(_topk_body, kk),
        grid=(n // p,),
        in_specs=[pl.BlockSpec((p, m), lambda i: (i, 0))],
        out_specs=pl.BlockSpec((p, kk), lambda i: (i, 0)),
        out_shape=jax.ShapeDtypeStruct((n, kk), jnp.int32),
    )(dists)


def _ksmallest(dists, kk):
    return jax.vmap(lambda d: _topk_one(d, kk))(dists)


def _fps_body(npoint, xt_ref, xs_ref, o_ref):
    n = xt_ref.shape[1]
    iota = jax.lax.broadcasted_iota(jnp.int32, (1, n), 1)
    xr = xt_ref[0:1, :]
    yr = xt_ref[1:2, :]
    zr = xt_ref[2:3, :]

    def body(i, carry):
        distance, far = carry
        o_ref[0, i] = far
        cx = xs_ref[0, far]
        cy = xs_ref[0, n + far]
        cz = xs_ref[0, 2 * n + far]
        d = (xr - cx) ** 2 + (yr - cy) ** 2 + (zr - cz) ** 2
        distance = jnp.minimum(distance, d)
        mx = jnp.max(distance)
        far = jnp.min(jnp.where(distance == mx, iota, n)).astype(jnp.int32)
        return distance, far

    jax.lax.fori_loop(0, npoint, body,
                      (jnp.full((1, n), 1e10, jnp.float32), jnp.int32(0)))


def _fps_one(xyz, npoint):
    xt = xyz.T
    return pl.pallas_call(
        functools.partial(_fps_body, npoint),
        in_specs=[
            pl.BlockSpec(memory_space=pltpu.VMEM),
            pl.BlockSpec(memory_space=pltpu.SMEM),
        ],
        out_specs=pl.BlockSpec(memory_space=pltpu.SMEM),
        out_shape=jax.ShapeDtypeStruct((1, npoint), jnp.int32),
    )(xt, xt.reshape(1, -1))


def _farthest_point_sample(xyz, npoint):
    return jax.vmap(lambda x: _fps_one(x, npoint))(xyz)[:, 0, :]


def _mm(a, b):
    return jnp.dot(a.astype(jnp.bfloat16), b.astype(jnp.bfloat16),
                   preferred_element_type=jnp.float32)


_KV_D = 2 * D_MODEL + 128


def _proj_body(f_ref, xyz_ref, w1_ref, b1_ref, wq_ref, wkv_ref,
               q_ref, kv_ref):
    x = _mm(f_ref[...], w1_ref[...]) + b1_ref[...]
    q_ref[...] = _mm(x, wq_ref[...])
    kv_ref[:, :2 * D_MODEL] = _mm(x, wkv_ref[...])
    n = xyz_ref.shape[0]
    xpad = jnp.concatenate(
        [xyz_ref[...], jnp.zeros((n, 125), jnp.float32)], axis=1)
    kv_ref[:, 2 * D_MODEL:] = xpad


def _proj_one(feats, xyz, w1, b1, wq, wkv):
    n = feats.shape[0]
    return pl.pallas_call(
        _proj_body,
        out_shape=(jax.ShapeDtypeStruct((n, D_MODEL), jnp.float32),
                   jax.ShapeDtypeStruct((n, _KV_D), jnp.float32)),
    )(feats, xyz, w1, b1, wq, wkv)


_SC_NW = 32


def _sc_gather_rows(table, indices):
    m = indices.shape[0]
    d = table.shape[1]
    if m % (8 * _SC_NW) != 0 or m < 16 * _SC_NW:
        return jnp.take(table, indices, axis=0)
    b_per_w = m // _SC_NW
    ch = min(64, b_per_w)
    while b_per_w % ch:
        ch -= 8
    nchunk = b_per_w // ch
    mesh = plsc.VectorSubcoreMesh(core_axis_name="c", subcore_axis_name="s")

    @functools.partial(
        pl.kernel, mesh=mesh,
        out_type=jax.ShapeDtypeStruct((m, d), table.dtype),
        scratch_types=[
            pltpu.VMEM((ch,), jnp.int32),
            pltpu.VMEM((ch, d), table.dtype),
            pltpu.SemaphoreType.DMA,
        ])
    def kern(table_hbm, idx_hbm, out_hbm, idx_v, rows_v, sem):
        wid = jax.lax.axis_index("s") * 2 + jax.lax.axis_index("c")
        base = wid * b_per_w

        @pl.loop(0, nchunk)
        def _(c):
            off = base + c * ch
            pltpu.sync_copy(idx_hbm.at[pl.ds(off, ch)], idx_v)
            pltpu.async_copy(table_hbm.at[idx_v], rows_v, sem).wait()
            pltpu.sync_copy(rows_v, out_hbm.at[pl.ds(off, ch)])

    return kern(table, indices)


def _pad128(x):
    c = x.shape[-1]
    pad = (-c) % 128
    if pad:
        x = jnp.concatenate([x, jnp.zeros(x.shape[:-1] + (pad,), x.dtype)], -1)
    return x


def _sc_gather_batched(table, idx):
    b, n, d = table.shape
    off = (jnp.arange(b, dtype=idx.dtype) * n).reshape((b,) + (1,) * (idx.ndim - 1))
    flat = (idx + off).reshape(-1)
    out = _sc_gather_rows(table.reshape(b * n, d), flat)
    return out.reshape(idx.shape + (d,))


def _attn_body(q_ref, kvg_ref, xyz_ref, pre_ref,
               wd1_ref, bd1_ref, wd2_ref, bd2_ref,
               wg1_ref, bg1_ref, wg2_ref, bg2_ref,
               wf_ref, bf_ref, o_ref):
    kk, p, _ = kvg_ref.shape
    dm = D_MODEL
    qb = q_ref[...]
    xb = xyz_ref[...]
    kvtab = kvg_ref[...].reshape(kk * p, _KV_D)
    ktab = kvtab[:, :dm]
    vtab = kvtab[:, dm:2 * dm]
    xgv = kvtab[:, 2 * dm:2 * dm + 3]
    dx = jnp.broadcast_to(xb[None], (kk, p, 3)).reshape(kk * p, 3) - xgv
    r1 = jnp.maximum(_mm(dx, wd1_ref[...]) + bd1_ref[...], 0.0)
    pos = _mm(r1, wd2_ref[...]) + bd2_ref[...]
    qrep = jnp.broadcast_to(qb[None], (kk, p, dm)).reshape(kk * p, dm)
    h = qrep - ktab + pos
    sl = (_mm(jnp.maximum(_mm(h, wg1_ref[...]) + bg1_ref[...], 0.0),
              wg2_ref[...]) + bg2_ref[...]) * _RSQRT_D
    pv = vtab + pos
    m = sl[0:p]
    for k in range(1, kk):
        m = jnp.maximum(m, sl[k * p:(k + 1) * p])
    den = jnp.zeros_like(m)
    acc = jnp.zeros_like(m)
    for k in range(kk):
        e = jnp.exp(sl[k * p:(k + 1) * p] - m)
        den = den + e
        acc = acc + e * pv[k * p:(k + 1) * p]
    res = acc / den
    o_ref[...] = _mm(res, wf_ref[...]) + bf_ref[...] + pre_ref[...]


def _attn_one(q, kvg, xyz, pre,
              wd1, bd1, wd2, bd2, wg1, bg1, wg2, bg2, wf, bf):
    n = q.shape[0]
    kk = kvg.shape[0]
    dout = pre.shape[-1]
    p = min(n, 128)
    grid = (n // p,)
    full = lambda i: (0, 0)
    return pl.pallas_call(
        _attn_body,
        grid=grid,
        in_specs=[
            pl.BlockSpec((p, D_MODEL), lambda i: (i, 0)),
            pl.BlockSpec((kk, p, _KV_D), lambda i: (0, i, 0)),
            pl.BlockSpec((p, 3), lambda i: (i, 0)),
            pl.BlockSpec((p, dout), lambda i: (i, 0)),
            pl.BlockSpec((3, D_MODEL), full),
            pl.BlockSpec((1, D_MODEL), full),
            pl.BlockSpec((D_MODEL, D_MODEL), full),
            pl.BlockSpec((1, D_MODEL), full),
            pl.BlockSpec((D_MODEL, D_MODEL), full),
            pl.BlockSpec((1, D_MODEL), full),
            pl.BlockSpec((D_MODEL, D_MODEL), full),
            pl.BlockSpec((1, D_MODEL), full),
            pl.BlockSpec((D_MODEL, dout), full),
            pl.BlockSpec((1, dout), full),
        ],
        out_specs=pl.BlockSpec((p, dout), lambda i: (i, 0)),
        out_shape=jax.ShapeDtypeStruct((n, dout), jnp.float32),
    )(q, kvg, xyz, pre,
      wd1, bd1, wd2, bd2, wg1, bg1, wg2, bg2, wf, bf)


def _row(v):
    return v.reshape(1, -1)


def _transformer_block(p, xyz, feats):
    b, n, _ = xyz.shape
    kk = min(KNN, n)
    dists = _square_distance(xyz, xyz)
    idx = _ksmallest(dists, kk)
    idx_t = jnp.swapaxes(idx, 1, 2)

    proj = jax.vmap(_proj_one, in_axes=(0, 0, None, None, None, None))
    wkv = jnp.concatenate([p['wk']['w'], p['wv']['w']], axis=1)
    q, kvt = proj(feats, xyz, p['fc1']['w'], _row(p['fc1']['b']),
                  p['wq']['w'], wkv)

    kvg = _sc_gather_batched(kvt, idx_t)

    attn = jax.vmap(
        _attn_one,
        in_axes=(0, 0, 0, 0) + (None,) * 10)
    out = attn(q, kvg, xyz, feats,
               p['d1']['w'], _row(p['d1']['b']),
               p['d2']['w'], _row(p['d2']['b']),
               p['g1']['w'], _row(p['g1']['b']),
               p['g2']['w'], _row(p['g2']['b']),
               p['fc2']['w'], _row(p['fc2']['b']))
    return out


def _transition_down(p, xyz, points, npoint, nsample):
    c = points.shape[-1]
    fps_idx = _farthest_point_sample(xyz, npoint)
    tbl = _pad128(jnp.concatenate([xyz, points], axis=-1))
    new_xyz = _sc_gather_batched(tbl, fps_idx)[..., :3]
    dists = _square_distance(new_xyz, xyz)
    idx = _ksmallest(dists, nsample)
    g = _sc_gather_batched(tbl, idx)
    grouped_norm = g[..., :3] - new_xyz[:, :, None, :]
    grouped_pts = g[..., 3:3 + c]
    h = jnp.concatenate([grouped_norm, grouped_pts], axis=-1)
    h = jax.nn.relu(_bn_train(_linear(p['c1'], h), p['bn1'], (0, 1, 2)))
    h = jax.nn.relu(_bn_train(_linear(p['c2'], h), p['bn2'], (0, 1, 2)))
    return new_xyz, jnp.max(h, axis=2)


def _transition_up(p, xyz1, points1, xyz2, points2):
    feats1 = jax.nn.relu(_bn_train(_linear(p['fc1'], points1), p['bn1'], (0, 1)))
    feats2 = jax.nn.relu(_bn_train(_linear(p['fc2'], points2), p['bn2'], (0, 1)))
    dists = _square_distance(xyz2, xyz1)
    idx = _ksmallest(dists, 3)
    d3 = jnp.take_along_axis(dists, idx, axis=-1)
    recip = 1.0 / (d3 + 1e-8)
    w = recip / jnp.sum(recip, -1, keepdims=True)
    c = feats1.shape[-1]
    rows = _sc_gather_batched(_pad128(feats1), idx)[..., :c]
    interp = jnp.sum(rows * w[..., None], axis=2)
    return interp + feats2


def _forward(params, x):
    nblocks = 4
    npts = x.shape[1]
    xyz = x[..., :3]
    h = _linear(params['bb_fc1b'], jax.nn.relu(_linear(params['bb_fc1a'], x)))
    points = _transformer_block(params['tf1'], xyz, h)
    xyz_and_feats = [(xyz, points)]
    for i in range(nblocks):
        xyz, points = _transition_down(params['td%d' % i], xyz, points,
                                       npts // 4 ** (i + 1), KNN)
        points = _transformer_block(params['bbtf%d' % i], xyz, points)
        xyz_and_feats.append((xyz, points))
    xyz = xyz_and_feats[-1][0]
    h = jax.nn.relu(_linear(params['f2a'], points))
    h = jax.nn.relu(_linear(params['f2b'], h))
    h = _linear(params['f2c'], h)
    points = _transformer_block(params['tf2'], xyz, h)
    for i in range(nblocks):
        points = _transition_up(params['tu%d' % i], xyz, points,
                                xyz_and_feats[-i - 2][0],
                                xyz_and_feats[-i - 2][1])
        xyz = xyz_and_feats[-i - 2][0]
        points = _transformer_block(params['uptf%d' % i], xyz, points)
    h = jax.nn.relu(_linear(params['f3a'], points))
    h = jax.nn.relu(_linear(params['f3b'], h))
    return _linear(params['f3c'], h)


def kernel(x, params):
    return _forward(params, x)


# speedup vs baseline: 1.3895x; 1.0078x over previous
import functools

import jax
import jax.numpy as jnp
import numpy as np
from jax.experimental import pallas as pl
from jax.experimental.pallas import tpu as pltpu
from jax.experimental.pallas import tpu_sc as plsc

D_MODEL = 512
KNN = 16
_RSQRT_D = 1.0 / np.sqrt(D_MODEL).astype(np.float32)


def _index_points(points, idx):
    b = points.shape[0]
    batch = jnp.arange(b).reshape((b,) + (1,) * (idx.ndim - 1))
    return points[batch, idx]


def _square_distance(src, dst):
    d = -2.0 * jnp.einsum('bnc,bmc->bnm', src, dst)
    d = d + jnp.sum(src ** 2, -1)[:, :, None]
    d = d + jnp.sum(dst ** 2, -1)[:, None, :]
    return d


def _bn_train(x, p, axes):
    m = jnp.mean(x, axis=axes, keepdims=True)
    v = jnp.var(x, axis=axes, keepdims=True)
    return (x - m) / jnp.sqrt(v + 1e-5) * p['g'] + p['b']


def _linear(p, x):
    y = x @ p['w']
    if 'b' in p:
        y = y + p['b']
    return y


def _topk_body(kk, d_ref, o_ref):
    d = d_ref[...]
    p, m = d.shape
    iota = jax.lax.broadcasted_iota(jnp.int32, (p, m), 1)
    col = jax.lax.broadcasted_iota(jnp.int32, (p, kk), 1)
    idxacc = jnp.zeros((p, kk), jnp.int32)
    for k in range(kk):
        mn = jnp.min(d, axis=1, keepdims=True)
        am = jnp.min(jnp.where(d == mn, iota, m), axis=1, keepdims=True)
        idxacc = jnp.where(col == k, am.astype(jnp.int32), idxacc)
        d = jnp.where(iota == am, jnp.inf, d)
    o_ref[...] = idxacc


def _topk_one(dists, kk):
    n, m = dists.shape
    p = min(n, 256)
    return pl.pallas_call(
        functools.partial(_topk_body, kk),
        grid=(n // p,),
        in_specs=[pl.BlockSpec((p, m), lambda i: (i, 0))],
        out_specs=pl.BlockSpec((p, kk), lambda i: (i, 0)),
        out_shape=jax.ShapeDtypeStruct((n, kk), jnp.int32),
    )(dists)


def _ksmallest(dists, kk):
    return jax.vmap(lambda d: _topk_one(d, kk))(dists)


def _fps_body(npoint, xt_ref, xs_ref, o_ref):
    _, rows, cols = xt_ref.shape
    n = rows * cols
    iota = (jax.lax.broadcasted_iota(jnp.int32, (rows, cols), 0) * cols
            + jax.lax.broadcasted_iota(jnp.int32, (rows, cols), 1))
    xr = xt_ref[0]
    yr = xt_ref[1]
    zr = xt_ref[2]

    def body(i, carry):
        distance, far = carry
        o_ref[0, i] = far
        cx = xs_ref[0, far]
        cy = xs_ref[0, n + far]
        cz = xs_ref[0, 2 * n + far]
        d = (xr - cx) ** 2 + (yr - cy) ** 2 + (zr - cz) ** 2
        distance = jnp.minimum(distance, d)
        mx = jnp.max(distance)
        far = jnp.min(jnp.where(distance == mx, iota, n)).astype(jnp.int32)
        return distance, far

    jax.lax.fori_loop(0, npoint, body,
                      (jnp.full((rows, cols), 1e10, jnp.float32),
                       jnp.int32(0)))


def _fps_one(xyz, npoint):
    xt = xyz.T
    n = xyz.shape[0]
    return pl.pallas_call(
        functools.partial(_fps_body, npoint),
        in_specs=[
            pl.BlockSpec(memory_space=pltpu.VMEM),
            pl.BlockSpec(memory_space=pltpu.SMEM),
        ],
        out_specs=pl.BlockSpec(memory_space=pltpu.SMEM),
        out_shape=jax.ShapeDtypeStruct((1, npoint), jnp.int32),
    )(xt.reshape(3, 8, n // 8), xt.reshape(1, -1))


def _farthest_point_sample(xyz, npoint):
    return jax.vmap(lambda x: _fps_one(x, npoint))(xyz)[:, 0, :]


def _mm(a, b):
    return jnp.dot(a.astype(jnp.bfloat16), b.astype(jnp.bfloat16),
                   preferred_element_type=jnp.float32)


_KV_D = 2 * D_MODEL + 128


def _proj_body(f_ref, xyz_ref, w1_ref, b1_ref, wq_ref, wkv_ref,
               q_ref, kv_ref):
    x = _mm(f_ref[...], w1_ref[...]) + b1_ref[...]
    q_ref[...] = _mm(x, wq_ref[...])
    kv_ref[:, :2 * D_MODEL] = _mm(x, wkv_ref[...])
    n = xyz_ref.shape[0]
    xpad = jnp.concatenate(
        [xyz_ref[...], jnp.zeros((n, 125), jnp.float32)], axis=1)
    kv_ref[:, 2 * D_MODEL:] = xpad


def _proj_one(feats, xyz, w1, b1, wq, wkv):
    n = feats.shape[0]
    return pl.pallas_call(
        _proj_body,
        out_shape=(jax.ShapeDtypeStruct((n, D_MODEL), jnp.float32),
                   jax.ShapeDtypeStruct((n, _KV_D), jnp.float32)),
    )(feats, xyz, w1, b1, wq, wkv)


_SC_NW = 32


def _sc_gather_rows(table, indices):
    m = indices.shape[0]
    d = table.shape[1]
    if m % (8 * _SC_NW) != 0 or m < 16 * _SC_NW:
        return jnp.take(table, indices, axis=0)
    b_per_w = m // _SC_NW
    ch = min(64, b_per_w)
    while b_per_w % ch:
        ch -= 8
    nchunk = b_per_w // ch
    mesh = plsc.VectorSubcoreMesh(core_axis_name="c", subcore_axis_name="s")

    @functools.partial(
        pl.kernel, mesh=mesh,
        out_type=jax.ShapeDtypeStruct((m, d), table.dtype),
        scratch_types=[
            pltpu.VMEM((ch,), jnp.int32),
            pltpu.VMEM((ch, d), table.dtype),
            pltpu.SemaphoreType.DMA,
        ])
    def kern(table_hbm, idx_hbm, out_hbm, idx_v, rows_v, sem):
        wid = jax.lax.axis_index("s") * 2 + jax.lax.axis_index("c")
        base = wid * b_per_w

        @pl.loop(0, nchunk)
        def _(c):
            off = base + c * ch
            pltpu.sync_copy(idx_hbm.at[pl.ds(off, ch)], idx_v)
            pltpu.async_copy(table_hbm.at[idx_v], rows_v, sem).wait()
            pltpu.sync_copy(rows_v, out_hbm.at[pl.ds(off, ch)])

    return kern(table, indices)


def _pad128(x):
    c = x.shape[-1]
    pad = (-c) % 128
    if pad:
        x = jnp.concatenate([x, jnp.zeros(x.shape[:-1] + (pad,), x.dtype)], -1)
    return x


def _sc_gather_batched(table, idx):
    b, n, d = table.shape
    off = (jnp.arange(b, dtype=idx.dtype) * n).reshape((b,) + (1,) * (idx.ndim - 1))
    flat = (idx + off).reshape(-1)
    out = _sc_gather_rows(table.reshape(b * n, d), flat)
    return out.reshape(idx.shape + (d,))


def _attn_body(q_ref, kvg_ref, xyz_ref, pre_ref,
               wd1_ref, bd1_ref, wd2_ref, bd2_ref,
               wg1_ref, bg1_ref, wg2_ref, bg2_ref,
               wf_ref, bf_ref, o_ref):
    kk, p, _ = kvg_ref.shape
    dm = D_MODEL
    qb = q_ref[...]
    xb = xyz_ref[...]
    kvtab = kvg_ref[...].reshape(kk * p, _KV_D)
    ktab = kvtab[:, :dm]
    vtab = kvtab[:, dm:2 * dm]
    xgv = kvtab[:, 2 * dm:2 * dm + 3]
    dx = jnp.broadcast_to(xb[None], (kk, p, 3)).reshape(kk * p, 3) - xgv
    r1 = jnp.maximum(_mm(dx, wd1_ref[...]) + bd1_ref[...], 0.0)
    pos = _mm(r1, wd2_ref[...]) + bd2_ref[...]
    qrep = jnp.broadcast_to(qb[None], (kk, p, dm)).reshape(kk * p, dm)
    h = qrep - ktab + pos
    sl = (_mm(jnp.maximum(_mm(h, wg1_ref[...]) + bg1_ref[...], 0.0),
              wg2_ref[...]) + bg2_ref[...]) * _RSQRT_D
    pv = vtab + pos
    m = sl[0:p]
    for k in range(1, kk):
        m = jnp.maximum(m, sl[k * p:(k + 1) * p])
    den = jnp.zeros_like(m)
    acc = jnp.zeros_like(m)
    for k in range(kk):
        e = jnp.exp(sl[k * p:(k + 1) * p] - m)
        den = den + e
        acc = acc + e * pv[k * p:(k + 1) * p]
    res = acc / den
    o_ref[...] = _mm(res, wf_ref[...]) + bf_ref[...] + pre_ref[...]


def _attn_one(q, kvg, xyz, pre,
              wd1, bd1, wd2, bd2, wg1, bg1, wg2, bg2, wf, bf):
    n = q.shape[0]
    kk = kvg.shape[0]
    dout = pre.shape[-1]
    p = min(n, 128)
    grid = (n // p,)
    full = lambda i: (0, 0)
    return pl.pallas_call(
        _attn_body,
        grid=grid,
        in_specs=[
            pl.BlockSpec((p, D_MODEL), lambda i: (i, 0)),
            pl.BlockSpec((kk, p, _KV_D), lambda i: (0, i, 0)),
            pl.BlockSpec((p, 3), lambda i: (i, 0)),
            pl.BlockSpec((p, dout), lambda i: (i, 0)),
            pl.BlockSpec((3, D_MODEL), full),
            pl.BlockSpec((1, D_MODEL), full),
            pl.BlockSpec((D_MODEL, D_MODEL), full),
            pl.BlockSpec((1, D_MODEL), full),
            pl.BlockSpec((D_MODEL, D_MODEL), full),
            pl.BlockSpec((1, D_MODEL), full),
            pl.BlockSpec((D_MODEL, D_MODEL), full),
            pl.BlockSpec((1, D_MODEL), full),
            pl.BlockSpec((D_MODEL, dout), full),
            pl.BlockSpec((1, dout), full),
        ],
        out_specs=pl.BlockSpec((p, dout), lambda i: (i, 0)),
        out_shape=jax.ShapeDtypeStruct((n, dout), jnp.float32),
    )(q, kvg, xyz, pre,
      wd1, bd1, wd2, bd2, wg1, bg1, wg2, bg2, wf, bf)


def _row(v):
    return v.reshape(1, -1)


def _transformer_block(p, xyz, feats):
    b, n, _ = xyz.shape
    kk = min(KNN, n)
    dists = _square_distance(xyz, xyz)
    idx = _ksmallest(dists, kk)
    idx_t = jnp.swapaxes(idx, 1, 2)

    proj = jax.vmap(_proj_one, in_axes=(0, 0, None, None, None, None))
    wkv = jnp.concatenate([p['wk']['w'], p['wv']['w']], axis=1)
    q, kvt = proj(feats, xyz, p['fc1']['w'], _row(p['fc1']['b']),
                  p['wq']['w'], wkv)

    kvg = _sc_gather_batched(kvt, idx_t)

    attn = jax.vmap(
        _attn_one,
        in_axes=(0, 0, 0, 0) + (None,) * 10)
    out = attn(q, kvg, xyz, feats,
               p['d1']['w'], _row(p['d1']['b']),
               p['d2']['w'], _row(p['d2']['b']),
               p['g1']['w'], _row(p['g1']['b']),
               p['g2']['w'], _row(p['g2']['b']),
               p['fc2']['w'], _row(p['fc2']['b']))
    return out


def _transition_down(p, xyz, points, npoint, nsample):
    c = points.shape[-1]
    fps_idx = _farthest_point_sample(xyz, npoint)
    tbl = _pad128(jnp.concatenate([xyz, points], axis=-1))
    new_xyz = _sc_gather_batched(tbl, fps_idx)[..., :3]
    dists = _square_distance(new_xyz, xyz)
    idx = _ksmallest(dists, nsample)
    g = _sc_gather_batched(tbl, idx)
    grouped_norm = g[..., :3] - new_xyz[:, :, None, :]
    grouped_pts = g[..., 3:3 + c]
    h = jnp.concatenate([grouped_norm, grouped_pts], axis=-1)
    h = jax.nn.relu(_bn_train(_linear(p['c1'], h), p['bn1'], (0, 1, 2)))
    h = jax.nn.relu(_bn_train(_linear(p['c2'], h), p['bn2'], (0, 1, 2)))
    return new_xyz, jnp.max(h, axis=2)


def _transition_up(p, xyz1, points1, xyz2, points2):
    feats1 = jax.nn.relu(_bn_train(_linear(p['fc1'], points1), p['bn1'], (0, 1)))
    feats2 = jax.nn.relu(_bn_train(_linear(p['fc2'], points2), p['bn2'], (0, 1)))
    dists = _square_distance(xyz2, xyz1)
    idx = _ksmallest(dists, 3)
    d3 = jnp.take_along_axis(dists, idx, axis=-1)
    recip = 1.0 / (d3 + 1e-8)
    w = recip / jnp.sum(recip, -1, keepdims=True)
    c = feats1.shape[-1]
    rows = _sc_gather_batched(_pad128(feats1), idx)[..., :c]
    interp = jnp.sum(rows * w[..., None], axis=2)
    return interp + feats2


def _forward(params, x):
    nblocks = 4
    npts = x.shape[1]
    xyz = x[..., :3]
    h = _linear(params['bb_fc1b'], jax.nn.relu(_linear(params['bb_fc1a'], x)))
    points = _transformer_block(params['tf1'], xyz, h)
    xyz_and_feats = [(xyz, points)]
    for i in range(nblocks):
        xyz, points = _transition_down(params['td%d' % i], xyz, points,
                                       npts // 4 ** (i + 1), KNN)
        points = _transformer_block(params['bbtf%d' % i], xyz, points)
        xyz_and_feats.append((xyz, points))
    xyz = xyz_and_feats[-1][0]
    h = jax.nn.relu(_linear(params['f2a'], points))
    h = jax.nn.relu(_linear(params['f2b'], h))
    h = _linear(params['f2c'], h)
    points = _transformer_block(params['tf2'], xyz, h)
    for i in range(nblocks):
        points = _transition_up(params['tu%d' % i], xyz, points,
                                xyz_and_feats[-i - 2][0],
                                xyz_and_feats[-i - 2][1])
        xyz = xyz_and_feats[-i - 2][0]
        points = _transformer_block(params['uptf%d' % i], xyz, points)
    h = jax.nn.relu(_linear(params['f3a'], points))
    h = jax.nn.relu(_linear(params['f3b'], h))
    return _linear(params['f3c'], h)


def kernel(x, params):
    return _forward(params, x)
